# Initial kernel scaffold; baseline (speedup 1.0000x reference)
import jax
import jax.numpy as jnp
from jax.experimental import pallas as pl


def kernel(hidden_states, weight):
    raise NotImplementedError("write your pallas kernel here")


# speedup vs baseline: 1.1213x; 1.1213x over previous
import functools

import jax
import jax.numpy as jnp
from jax.experimental import pallas as pl
from jax.experimental.pallas import tpu as pltpu

NUM_EXPERTS = 64
TOP_K = 8
HIDDEN = 4096
TOKENS = 32768

TOKEN_BLOCK = 512


def _router_block_kernel(x_ref, w_ref, probs_ref, scores_ref, idx_ref):
    x = x_ref[...]
    w = w_ref[...]
    logits = jax.lax.dot_general(
        x, w,
        dimension_numbers=(((1,), (1,)), ((), ())),
        preferred_element_type=jnp.float32,
    )

    m = jnp.max(logits, axis=-1, keepdims=True)
    e = jnp.exp(logits - m)
    denom = jnp.sum(e, axis=-1, keepdims=True)
    probs = e / denom
    probs_ref[...] = probs

    t = probs.shape[0]
    lane_iota = jax.lax.broadcasted_iota(jnp.int32, (t, NUM_EXPERTS), 1)
    work = probs
    vals = []
    idxs = []
    for _ in range(TOP_K):
        cur = jnp.max(work, axis=-1, keepdims=True)
        hit = work == cur
        cur_idx = jnp.min(
            jnp.where(hit, lane_iota, NUM_EXPERTS), axis=-1, keepdims=True
        )
        vals.append(cur)
        idxs.append(cur_idx)
        work = jnp.where(lane_iota == cur_idx, -1.0, work)

    top_vals = jnp.concatenate(vals, axis=-1)
    top_idx = jnp.concatenate(idxs, axis=-1)
    scores_ref[...] = top_vals / jnp.sum(top_vals, axis=-1, keepdims=True)
    idx_ref[...] = top_idx


@jax.jit
def kernel(hidden_states, weight):
    n_tokens = hidden_states.shape[0]
    grid = (n_tokens // TOKEN_BLOCK,)
    probs, scores, idx = pl.pallas_call(
        _router_block_kernel,
        grid=grid,
        in_specs=[
            pl.BlockSpec((TOKEN_BLOCK, HIDDEN), lambda i: (i, 0)),
            pl.BlockSpec((NUM_EXPERTS, HIDDEN), lambda i: (0, 0)),
        ],
        out_specs=[
            pl.BlockSpec((TOKEN_BLOCK, NUM_EXPERTS), lambda i: (i, 0)),
            pl.BlockSpec((TOKEN_BLOCK, TOP_K), lambda i: (i, 0)),
            pl.BlockSpec((TOKEN_BLOCK, TOP_K), lambda i: (i, 0)),
        ],
        out_shape=[
            jax.ShapeDtypeStruct((n_tokens, NUM_EXPERTS), jnp.float32),
            jax.ShapeDtypeStruct((n_tokens, TOP_K), jnp.float32),
            jax.ShapeDtypeStruct((n_tokens, TOP_K), jnp.int32),
        ],
    )(hidden_states, weight)
    return (probs, scores, idx)


# speedup vs baseline: 1.3499x; 1.2038x over previous
import functools

import jax
import jax.numpy as jnp
from jax.experimental import pallas as pl
from jax.experimental.pallas import tpu as pltpu

NUM_EXPERTS = 64
TOP_K = 8
HIDDEN = 4096
TOKENS = 32768

TOKEN_BLOCK = 512


def _router_block_kernel(x_ref, w_ref, probs_ref, scores_ref, idx_ref):
    x = x_ref[...]
    w = w_ref[...]
    logits = jax.lax.dot_general(
        x, w,
        dimension_numbers=(((1,), (1,)), ((), ())),
        preferred_element_type=jnp.float32,
    )

    m = jnp.max(logits, axis=-1, keepdims=True)
    e = jnp.exp(logits - m)
    denom = jnp.sum(e, axis=-1, keepdims=True)
    probs = e / denom
    probs_ref[...] = probs

    t = probs.shape[0]
    lane_iota = jax.lax.broadcasted_iota(jnp.int32, (t, NUM_EXPERTS), 1)
    bits = jax.lax.bitcast_convert_type(probs, jnp.int32)
    keys = jax.lax.bitcast_convert_type(
        (bits & ~63) | (63 - lane_iota), jnp.float32
    )
    work = keys
    tops = []
    for _ in range(TOP_K):
        cur = jnp.max(work, axis=-1, keepdims=True)
        tops.append(cur)
        work = jnp.where(work == cur, -1.0, work)

    top_keys = jax.lax.bitcast_convert_type(
        jnp.concatenate(tops, axis=-1), jnp.int32
    )
    top_idx = 63 - (top_keys & 63)
    top_vals = jax.lax.bitcast_convert_type(top_keys & ~63, jnp.float32)
    scores_ref[...] = top_vals / jnp.sum(top_vals, axis=-1, keepdims=True)
    idx_ref[...] = top_idx


@jax.jit
def kernel(hidden_states, weight):
    n_tokens = hidden_states.shape[0]
    grid = (n_tokens // TOKEN_BLOCK,)
    probs, scores, idx = pl.pallas_call(
        _router_block_kernel,
        grid=grid,
        in_specs=[
            pl.BlockSpec((TOKEN_BLOCK, HIDDEN), lambda i: (i, 0)),
            pl.BlockSpec((NUM_EXPERTS, HIDDEN), lambda i: (0, 0)),
        ],
        out_specs=[
            pl.BlockSpec((TOKEN_BLOCK, NUM_EXPERTS), lambda i: (i, 0)),
            pl.BlockSpec((TOKEN_BLOCK, TOP_K), lambda i: (i, 0)),
            pl.BlockSpec((TOKEN_BLOCK, TOP_K), lambda i: (i, 0)),
        ],
        out_shape=[
            jax.ShapeDtypeStruct((n_tokens, NUM_EXPERTS), jnp.float32),
            jax.ShapeDtypeStruct((n_tokens, TOP_K), jnp.float32),
            jax.ShapeDtypeStruct((n_tokens, TOP_K), jnp.int32),
        ],
    )(hidden_states, weight)
    return (probs, scores, idx)
